# baseline (speedup 1.0000x reference)
import functools

import jax
import jax.numpy as jnp
from jax import lax
from jax.experimental import pallas as pl
from jax.experimental.pallas import tpu as pltpu
from jax.experimental.pallas import tpu_sc as plsc

B = 8
N = 16384
S = 512
K = 16
C = 6
L = 16
NCHUNK = N // L
WPB = 4
ROWS_PER_W = S // WPB
HALF_ROWS = ROWS_PER_W // 2
STAGE = HALF_ROWS * K * C
BIG_F = 3.0e38
BIG_I = 2147483647


def _body(x_hbm, out_hbm, xb_v, dist_v, knn_v, stage_v):
    iota = lax.iota(jnp.int32, L)

    cid = lax.axis_index("c")
    sid = lax.axis_index("s")
    wid = sid * 2 + cid
    b = wid // WPB
    j = wid % WPB
    row_lo = j * ROWS_PER_W

    pltpu.sync_copy(x_hbm.at[b], xb_v)

    def splat_f(v):
        return jnp.full((L,), v, dtype=jnp.float32)

    def splat_i(v):
        return jnp.full((L,), v, dtype=jnp.int32)

    def init_chunk(ch, _):
        plsc.store_scatter(dist_v, [iota + ch * L], splat_f(1e10))
        return 0
    lax.fori_loop(0, NCHUNK, init_chunk, 0)

    iota6 = iota * C
    U = 8

    def fps_iter(i, farthest):
        fidx6 = splat_i(farthest * C)
        cx = plsc.load_gather(xb_v, [fidx6])
        cy = plsc.load_gather(xb_v, [fidx6 + 1])
        cz = plsc.load_gather(xb_v, [fidx6 + 2])
        is_owner = jnp.logical_and(i >= row_lo, i < row_lo + ROWS_PER_W)

        def dist_chunk(g, u):
            pidx = iota + (g * (U * L) + u * L)
            pidx6 = iota6 + (g * (U * L * C) + u * (L * C))
            px = plsc.load_gather(xb_v, [pidx6])
            py = plsc.load_gather(xb_v, [pidx6 + 1])
            pz = plsc.load_gather(xb_v, [pidx6 + 2])
            dx = px - cx
            dy = py - cy
            dz = pz - cz
            return pidx, dx * dx + dy * dy + dz * dz

        def light_group(g, carry):
            avs, ais = carry
            avs, ais = list(avs), list(ais)
            for u in range(U):
                pidx, d = dist_chunk(g, u)
                dold = plsc.load_gather(dist_v, [pidx])
                dnew = jnp.minimum(dold, d)
                plsc.store_scatter(dist_v, [pidx], dnew)
                m = dnew > avs[u]
                avs[u] = jnp.where(m, dnew, avs[u])
                ais[u] = jnp.where(m, pidx, ais[u])
            return tuple(avs), tuple(ais)

        avs, ais = lax.fori_loop(
            0, NCHUNK // U, light_group,
            (tuple(splat_f(-1.0) for _ in range(U)),
             tuple(splat_i(0) for _ in range(U))))

        rmax, ridx = avs[0], ais[0]
        for u in range(1, U):
            take = jnp.logical_or(
                avs[u] > rmax,
                jnp.logical_and(avs[u] == rmax, ais[u] < ridx))
            rmax = jnp.where(take, avs[u], rmax)
            ridx = jnp.where(take, ais[u], ridx)

        @pl.when(is_owner)
        def _():
            def merge_one(d, pidx, tv, ti):
                cs, ci = plsc.sort_key_val(d, pidx)
                csr = jnp.flip(cs, 0)
                cir = jnp.flip(ci, 0)
                take = csr < tv
                mv = jnp.where(take, csr, tv)
                mi = jnp.where(take, cir, ti)
                tv2, ti2 = plsc.sort_key_val(mv, mi)
                return tv2, ti2, splat_f(jnp.max(tv2))

            def topk_group(g, carry):
                tv, ti, kth = carry
                ds = []
                for u in range(U):
                    ds.append(dist_chunk(g, u))
                hit = jnp.zeros((L,), jnp.bool_)
                for u in range(U):
                    hit = jnp.logical_or(hit, ds[u][1] < kth)

                def do_merges(args):
                    tv, ti, kth = args
                    for u in range(U):
                        pidx, d = ds[u]

                        def m1(a, d=d, pidx=pidx):
                            return merge_one(d, pidx, a[0], a[1])

                        tv, ti, kth = lax.cond(
                            jnp.any(d < kth), m1, lambda a: a, (tv, ti, kth))
                    return tv, ti, kth

                return lax.cond(
                    jnp.any(hit), do_merges, lambda a: a, (tv, ti, kth))

            _, topi, _ = lax.fori_loop(
                0, NCHUNK // U, topk_group,
                (splat_f(BIG_F), splat_i(0), splat_f(BIG_F)))
            plsc.store_scatter(knn_v, [iota + (i - row_lo) * K], topi)

        mval = jnp.max(rmax)
        cand = jnp.where(rmax == splat_f(mval), ridx, splat_i(BIG_I))
        return jnp.min(cand)

    lax.fori_loop(0, S, fps_iter, jnp.int32(0))

    out_base = b * (S * K * C) + j * (ROWS_PER_W * K * C)
    for half in range(2):
        def gather_row(r, _):
            row = half * HALF_ROWS + r
            idxv = plsc.load_gather(knn_v, [iota + row * K])
            sbase = r * (K * C)
            idx6 = idxv * C
            for c in range(C):
                vals = plsc.load_gather(xb_v, [idx6 + c])
                plsc.store_scatter(stage_v, [iota * C + (sbase + c)], vals)
            return 0
        lax.fori_loop(0, HALF_ROWS, gather_row, 0)
        pltpu.sync_copy(
            stage_v, out_hbm.at[pl.ds(out_base + half * STAGE, STAGE)])


@jax.jit
def kernel(x):
    mesh = plsc.VectorSubcoreMesh(core_axis_name="c", subcore_axis_name="s")
    run = pl.kernel(
        _body,
        out_type=jax.ShapeDtypeStruct((B * S * K * C,), jnp.float32),
        mesh=mesh,
        compiler_params=pltpu.CompilerParams(needs_layout_passes=False),
        scratch_types=[
            pltpu.VMEM((N * C,), jnp.float32),
            pltpu.VMEM((N,), jnp.float32),
            pltpu.VMEM((ROWS_PER_W * K,), jnp.int32),
            pltpu.VMEM((STAGE,), jnp.float32),
        ],
    )
    return run(x.reshape(B, N * C)).reshape(B, S, K, C)


# speedup vs baseline: 1.0134x; 1.0134x over previous
import functools

import jax
import jax.numpy as jnp
from jax import lax
from jax.experimental import pallas as pl
from jax.experimental.pallas import tpu as pltpu
from jax.experimental.pallas import tpu_sc as plsc

B = 8
N = 16384
S = 512
K = 16
C = 6
L = 16
NCHUNK = N // L
WPB = 4
ROWS_PER_W = S // WPB
HALF_ROWS = ROWS_PER_W // 2
STAGE = HALF_ROWS * K * C
BIG_F = 3.0e38
BIG_I = 2147483647


def _body(x_hbm, out_hbm, xb_v, dist_v, knn_v, stage_v):
    iota = lax.iota(jnp.int32, L)

    cid = lax.axis_index("c")
    sid = lax.axis_index("s")
    wid = sid * 2 + cid
    b = wid // WPB
    j = wid % WPB
    row_lo = j * ROWS_PER_W

    pltpu.sync_copy(x_hbm.at[b], xb_v)

    def splat_f(v):
        return jnp.full((L,), v, dtype=jnp.float32)

    def splat_i(v):
        return jnp.full((L,), v, dtype=jnp.int32)

    def init_chunk(ch, _):
        plsc.store_scatter(dist_v, [iota + ch * L], splat_f(1e10))
        return 0
    lax.fori_loop(0, NCHUNK, init_chunk, 0)

    iota6 = iota * C
    U = 4

    def fps_iter(i, farthest):
        fidx6 = splat_i(farthest * C)
        cx = plsc.load_gather(xb_v, [fidx6])
        cy = plsc.load_gather(xb_v, [fidx6 + 1])
        cz = plsc.load_gather(xb_v, [fidx6 + 2])
        is_owner = jnp.logical_and(i >= row_lo, i < row_lo + ROWS_PER_W)

        def dist_chunk(g, u):
            pidx = iota + (g * (U * L) + u * L)
            pidx6 = iota6 + (g * (U * L * C) + u * (L * C))
            px = plsc.load_gather(xb_v, [pidx6])
            py = plsc.load_gather(xb_v, [pidx6 + 1])
            pz = plsc.load_gather(xb_v, [pidx6 + 2])
            dx = px - cx
            dy = py - cy
            dz = pz - cz
            return pidx, dx * dx + dy * dy + dz * dz

        def light_group(g, carry):
            avs, ais = carry
            avs, ais = list(avs), list(ais)
            for u in range(U):
                pidx, d = dist_chunk(g, u)
                dold = plsc.load_gather(dist_v, [pidx])
                dnew = jnp.minimum(dold, d)
                plsc.store_scatter(dist_v, [pidx], dnew)
                m = dnew > avs[u]
                avs[u] = jnp.where(m, dnew, avs[u])
                ais[u] = jnp.where(m, pidx, ais[u])
            return tuple(avs), tuple(ais)

        avs, ais = lax.fori_loop(
            0, NCHUNK // U, light_group,
            (tuple(splat_f(-1.0) for _ in range(U)),
             tuple(splat_i(0) for _ in range(U))))

        rmax, ridx = avs[0], ais[0]
        for u in range(1, U):
            take = jnp.logical_or(
                avs[u] > rmax,
                jnp.logical_and(avs[u] == rmax, ais[u] < ridx))
            rmax = jnp.where(take, avs[u], rmax)
            ridx = jnp.where(take, ais[u], ridx)

        @pl.when(is_owner)
        def _():
            def merge_one(d, pidx, tv, ti):
                cs, ci = plsc.sort_key_val(d, pidx)
                csr = jnp.flip(cs, 0)
                cir = jnp.flip(ci, 0)
                take = csr < tv
                mv = jnp.where(take, csr, tv)
                mi = jnp.where(take, cir, ti)
                tv2, ti2 = plsc.sort_key_val(mv, mi)
                return tv2, ti2, splat_f(jnp.max(tv2))

            def topk_group(g, carry):
                tv, ti, kth = carry
                ds = []
                for u in range(U):
                    ds.append(dist_chunk(g, u))
                hit = jnp.zeros((L,), jnp.bool_)
                for u in range(U):
                    hit = jnp.logical_or(hit, ds[u][1] < kth)

                def do_merges(args):
                    tv, ti, kth = args
                    for u in range(U):
                        pidx, d = ds[u]

                        def m1(a, d=d, pidx=pidx):
                            return merge_one(d, pidx, a[0], a[1])

                        tv, ti, kth = lax.cond(
                            jnp.any(d < kth), m1, lambda a: a, (tv, ti, kth))
                    return tv, ti, kth

                return lax.cond(
                    jnp.any(hit), do_merges, lambda a: a, (tv, ti, kth))

            _, topi, _ = lax.fori_loop(
                0, NCHUNK // U, topk_group,
                (splat_f(BIG_F), splat_i(0), splat_f(BIG_F)))
            plsc.store_scatter(knn_v, [iota + (i - row_lo) * K], topi)

        mval = jnp.max(rmax)
        cand = jnp.where(rmax == splat_f(mval), ridx, splat_i(BIG_I))
        return jnp.min(cand)

    lax.fori_loop(0, S, fps_iter, jnp.int32(0))

    out_base = b * (S * K * C) + j * (ROWS_PER_W * K * C)
    for half in range(2):
        def gather_row(r, _):
            row = half * HALF_ROWS + r
            idxv = plsc.load_gather(knn_v, [iota + row * K])
            sbase = r * (K * C)
            idx6 = idxv * C
            for c in range(C):
                vals = plsc.load_gather(xb_v, [idx6 + c])
                plsc.store_scatter(stage_v, [iota * C + (sbase + c)], vals)
            return 0
        lax.fori_loop(0, HALF_ROWS, gather_row, 0)
        pltpu.sync_copy(
            stage_v, out_hbm.at[pl.ds(out_base + half * STAGE, STAGE)])


@jax.jit
def kernel(x):
    mesh = plsc.VectorSubcoreMesh(core_axis_name="c", subcore_axis_name="s")
    run = pl.kernel(
        _body,
        out_type=jax.ShapeDtypeStruct((B * S * K * C,), jnp.float32),
        mesh=mesh,
        compiler_params=pltpu.CompilerParams(needs_layout_passes=False),
        scratch_types=[
            pltpu.VMEM((N * C,), jnp.float32),
            pltpu.VMEM((N,), jnp.float32),
            pltpu.VMEM((ROWS_PER_W * K,), jnp.int32),
            pltpu.VMEM((STAGE,), jnp.float32),
        ],
    )
    return run(x.reshape(B, N * C)).reshape(B, S, K, C)
